# Initial kernel scaffold; baseline (speedup 1.0000x reference)
import jax
import jax.numpy as jnp
from jax.experimental import pallas as pl


def kernel(node_values, edges, W, b):
    raise NotImplementedError("write your pallas kernel here")


# speedup vs baseline: 5.1293x; 5.1293x over previous
import functools

import jax
import jax.numpy as jnp
from jax import lax
from jax.experimental import pallas as pl
from jax.experimental.pallas import tpu as pltpu
from jax.experimental.pallas import tpu_sc as plsc

V = 10000
D = 128
T = 4
E = 80000

NC = 2
NS = 16
CHUNK = 128
NCHUNKS = E // CHUNK
TYPES_PER_SC = T // NC

ROWS_MAIN = 624
ROWS_TAIL = V - ROWS_MAIN * NS
ZROWS = 208


def _sc_message_passing(node_values, edges_r):
    mesh = plsc.VectorSubcoreMesh(core_axis_name="c", subcore_axis_name="s")

    @functools.partial(
        pl.kernel,
        out_type=jax.ShapeDtypeStruct((T, V, D), jnp.float32),
        mesh=mesh,
        scratch_types=[
            pltpu.VMEM_SHARED((V, D), jnp.float32),
            pltpu.VMEM((ZROWS, D), jnp.float32),
            pltpu.VMEM((1, CHUNK), jnp.int32),
            pltpu.VMEM((1, CHUNK), jnp.int32),
            pltpu.VMEM((CHUNK, D), jnp.float32),
        ],
    )
    def sc_kernel(node_hbm, edges_hbm, msgs_hbm, acc, zbuf, src_idx, dst_idx,
                  rows):
        c = lax.axis_index("c")
        s = lax.axis_index("s")
        base = s * ROWS_MAIN

        @pl.loop(0, ZROWS)
        def _(r):
            @pl.loop(0, D // 16)
            def _(k):
                zbuf[r, pl.ds(k * 16, 16)] = jnp.zeros((16,), jnp.float32)

        for tt in range(TYPES_PER_SC):
            t = c * TYPES_PER_SC + tt

            @pl.loop(0, ROWS_MAIN // ZROWS)
            def _(z):
                pltpu.sync_copy(zbuf, acc.at[pl.ds(base + z * ZROWS, ZROWS)])

            @pl.when(s == NS - 1)
            def _():
                pltpu.sync_copy(zbuf.at[pl.ds(0, ROWS_TAIL)],
                                acc.at[pl.ds(ROWS_MAIN * NS, ROWS_TAIL)])

            plsc.subcore_barrier()

            nj = jnp.where(s < (NCHUNKS % NS), NCHUNKS // NS + 1, NCHUNKS // NS)

            @pl.loop(0, nj)
            def _(j):
                g = s + NS * j
                pltpu.sync_copy(edges_hbm.at[t, 0, g], src_idx)
                pltpu.sync_copy(edges_hbm.at[t, 1, g], dst_idx)
                pltpu.sync_copy(node_hbm.at[src_idx.at[0]], rows)
                pltpu.sync_copy(rows, acc.at[dst_idx.at[0]], add=True)

            plsc.subcore_barrier()

            pltpu.sync_copy(acc.at[pl.ds(base, ROWS_MAIN)],
                            msgs_hbm.at[t, pl.ds(base, ROWS_MAIN)])

            @pl.when(s == NS - 1)
            def _():
                pltpu.sync_copy(
                    acc.at[pl.ds(ROWS_MAIN * NS, ROWS_TAIL)],
                    msgs_hbm.at[t, pl.ds(ROWS_MAIN * NS, ROWS_TAIL)])

            plsc.subcore_barrier()

    return sc_kernel(node_values, edges_r)


BV = 2000


def _mm_body(msgs_ref, w_ref, b_ref, out_ref):
    t = pl.program_id(1)

    @pl.when(t == 0)
    def _():
        out_ref[...] = jnp.broadcast_to(b_ref[...], out_ref.shape)

    out_ref[...] += jnp.dot(msgs_ref[0], w_ref[0],
                            preferred_element_type=jnp.float32)


def _tc_matmul(msgs, w3, b2):
    return pl.pallas_call(
        _mm_body,
        grid=(V // BV, T),
        in_specs=[
            pl.BlockSpec((1, BV, D), lambda i, t: (t, i, 0)),
            pl.BlockSpec((1, D, D), lambda i, t: (t, 0, 0)),
            pl.BlockSpec((1, D), lambda i, t: (0, 0)),
        ],
        out_specs=pl.BlockSpec((BV, D), lambda i, t: (i, 0)),
        out_shape=jax.ShapeDtypeStruct((V, D), jnp.float32),
        compiler_params=pltpu.CompilerParams(
            dimension_semantics=("parallel", "arbitrary")),
    )(msgs, w3, b2)


def kernel(node_values, edges, W, b):
    edges_r = edges.astype(jnp.int32).reshape(T, 2, NCHUNKS, 1, CHUNK)
    msgs = _sc_message_passing(node_values, edges_r)
    return _tc_matmul(msgs, W.reshape(T, D, D), b.reshape(1, D))


# speedup vs baseline: 9.0444x; 1.7633x over previous
import functools

import jax
import jax.numpy as jnp
from jax import lax
from jax.experimental import pallas as pl
from jax.experimental.pallas import tpu as pltpu
from jax.experimental.pallas import tpu_sc as plsc

V = 10000
D = 128
T = 4
E = 80000

NC = 2
NS = 16
CHUNK = 128
NCHUNKS = E // CHUNK
TYPES_PER_SC = T // NC
NJ = NCHUNKS // NS
NJMAX = NJ + 1
NBUF = 2

ROWS_MAIN = 624
ROWS_TAIL = V - ROWS_MAIN * NS


def _sc_message_passing(node_values, edges_r, zeros):
    mesh = plsc.VectorSubcoreMesh(core_axis_name="c", subcore_axis_name="s")

    @functools.partial(
        pl.kernel,
        out_type=jax.ShapeDtypeStruct((T, V, D), jnp.float32),
        mesh=mesh,
        scratch_types=[
            pltpu.VMEM_SHARED((V, D), jnp.float32),
            pltpu.VMEM((NJMAX, 1, CHUNK), jnp.int32),
            pltpu.VMEM((NJMAX, 1, CHUNK), jnp.int32),
            pltpu.VMEM((NBUF, CHUNK, D), jnp.float32),
            pltpu.SemaphoreType.DMA((NBUF,)),
            pltpu.SemaphoreType.DMA((NBUF,)),
        ],
    )
    def sc_kernel(node_hbm, edges_hbm, zeros_hbm, msgs_hbm, acc, sidx, didx,
                  rows, gsem, ssem):
        c = lax.axis_index("c")
        s = lax.axis_index("s")
        base = s * ROWS_MAIN
        nj = jnp.where(s == NS - 1, NJMAX, NJ)

        def gather(j, r):
            pltpu.async_copy(node_hbm.at[sidx.at[j, 0]], rows.at[r],
                             gsem.at[r])

        def gather_wait(j, r):
            pltpu.make_async_copy(node_hbm.at[sidx.at[j, 0]], rows.at[r],
                                  gsem.at[r]).wait()

        def scatter(j, r):
            pltpu.async_copy(rows.at[r], acc.at[didx.at[j, 0]],
                             ssem.at[r], add=True)

        def scatter_wait(j, r):
            pltpu.make_async_copy(rows.at[r], acc.at[didx.at[j, 0]],
                                  ssem.at[r]).wait()

        for tt in range(TYPES_PER_SC):
            t = c * TYPES_PER_SC + tt

            pltpu.sync_copy(zeros_hbm, acc.at[pl.ds(base, ROWS_MAIN)])

            @pl.when(s == NS - 1)
            def _():
                pltpu.sync_copy(zeros_hbm.at[pl.ds(0, ROWS_TAIL)],
                                acc.at[pl.ds(ROWS_MAIN * NS, ROWS_TAIL)])

            pltpu.sync_copy(edges_hbm.at[t, 0, pl.ds(NJ * s, NJMAX)], sidx)
            pltpu.sync_copy(edges_hbm.at[t, 1, pl.ds(NJ * s, NJMAX)], didx)

            plsc.subcore_barrier()

            for r in range(NBUF):
                gather(r, r)

            @pl.loop(0, pl.cdiv(NJMAX, NBUF))
            def _(kk):
                for r in range(NBUF):
                    j = kk * NBUF + r

                    @pl.when(j < nj)
                    def _():
                        gather_wait(j, r)
                        scatter(j, r)

                    @pl.when(j + NBUF < nj)
                    def _():
                        scatter_wait(j, r)
                        gather(j + NBUF, r)

            for r in range(NBUF):
                scatter_wait(0, r)

            plsc.subcore_barrier()

            pltpu.sync_copy(acc.at[pl.ds(base, ROWS_MAIN)],
                            msgs_hbm.at[t, pl.ds(base, ROWS_MAIN)])

            @pl.when(s == NS - 1)
            def _():
                pltpu.sync_copy(
                    acc.at[pl.ds(ROWS_MAIN * NS, ROWS_TAIL)],
                    msgs_hbm.at[t, pl.ds(ROWS_MAIN * NS, ROWS_TAIL)])

            plsc.subcore_barrier()

    return sc_kernel(node_values, edges_r, zeros)


BV = 2000


def _mm_body(msgs_ref, w_ref, b_ref, out_ref):
    t = pl.program_id(1)

    @pl.when(t == 0)
    def _():
        out_ref[...] = jnp.broadcast_to(b_ref[...], out_ref.shape)

    out_ref[...] += jnp.dot(msgs_ref[0], w_ref[0],
                            preferred_element_type=jnp.float32)


def _tc_matmul(msgs, w3, b2):
    return pl.pallas_call(
        _mm_body,
        grid=(V // BV, T),
        in_specs=[
            pl.BlockSpec((1, BV, D), lambda i, t: (t, i, 0)),
            pl.BlockSpec((1, D, D), lambda i, t: (t, 0, 0)),
            pl.BlockSpec((1, D), lambda i, t: (0, 0)),
        ],
        out_specs=pl.BlockSpec((BV, D), lambda i, t: (i, 0)),
        out_shape=jax.ShapeDtypeStruct((V, D), jnp.float32),
        compiler_params=pltpu.CompilerParams(
            dimension_semantics=("parallel", "arbitrary")),
    )(msgs, w3, b2)


def kernel(node_values, edges, W, b):
    edges_r = edges.astype(jnp.int32).reshape(T, 2, NCHUNKS, 1, CHUNK)
    zeros = jnp.zeros((ROWS_MAIN, D), jnp.float32)
    msgs = _sc_message_passing(node_values, edges_r, zeros)
    return _tc_matmul(msgs, W.reshape(T, D, D), b.reshape(1, D))


# speedup vs baseline: 9.6408x; 1.0659x over previous
import functools

import jax
import jax.numpy as jnp
from jax import lax
from jax.experimental import pallas as pl
from jax.experimental.pallas import tpu as pltpu
from jax.experimental.pallas import tpu_sc as plsc

V = 10000
D = 128
T = 4
E = 80000

NC = 2
NS = 16
CHUNK = 128
NCHUNKS = E // CHUNK
TYPES_PER_SC = T // NC
NJ = NCHUNKS // NS
NJMAX = NJ + 1
NBUF = 2
ZROWS = 52

ROWS_MAIN = 624
ROWS_TAIL = V - ROWS_MAIN * NS


def _sc_message_passing(node_values, edges_r, zeros):
    mesh = plsc.VectorSubcoreMesh(core_axis_name="c", subcore_axis_name="s")

    @functools.partial(
        pl.kernel,
        out_type=jax.ShapeDtypeStruct((T, V, D), jnp.float32),
        mesh=mesh,
        scratch_types=[
            pltpu.VMEM_SHARED((V, D), jnp.float32),
            pltpu.VMEM((NJMAX, 1, CHUNK), jnp.int32),
            pltpu.VMEM((NJMAX, 1, CHUNK), jnp.int32),
            pltpu.VMEM((NBUF, CHUNK, D), jnp.float32),
            pltpu.VMEM((ZROWS, D), jnp.float32),
            pltpu.SemaphoreType.DMA((NBUF,)),
            pltpu.SemaphoreType.DMA((NBUF,)),
            pltpu.SemaphoreType.DMA,
        ],
    )
    def sc_kernel(node_hbm, edges_hbm, zeros_hbm, msgs_hbm, acc, sidx, didx,
                  rows, zbuf, gsem, ssem, zsem):
        c = lax.axis_index("c")
        s = lax.axis_index("s")
        base = s * ROWS_MAIN
        nj = jnp.where(s == NS - 1, NJMAX, NJ)

        pltpu.sync_copy(zeros_hbm, zbuf)

        def gather(j, r):
            pltpu.async_copy(node_hbm.at[sidx.at[j, 0]], rows.at[r],
                             gsem.at[r])

        def gather_wait(j, r):
            pltpu.make_async_copy(node_hbm.at[sidx.at[j, 0]], rows.at[r],
                                  gsem.at[r]).wait()

        def scatter(j, r):
            pltpu.async_copy(rows.at[r], acc.at[didx.at[j, 0]],
                             ssem.at[r], add=True)

        def scatter_wait(j, r):
            pltpu.make_async_copy(rows.at[r], acc.at[didx.at[j, 0]],
                                  ssem.at[r]).wait()

        for tt in range(TYPES_PER_SC):
            t = c * TYPES_PER_SC + tt

            for k in range(ROWS_MAIN // ZROWS):
                pltpu.async_copy(zbuf, acc.at[pl.ds(base + k * ZROWS, ZROWS)],
                                 zsem)

            @pl.when(s == NS - 1)
            def _():
                pltpu.async_copy(zbuf.at[pl.ds(0, ROWS_TAIL)],
                                 acc.at[pl.ds(ROWS_MAIN * NS, ROWS_TAIL)],
                                 zsem)

            pltpu.sync_copy(edges_hbm.at[t, 0, pl.ds(NJ * s, NJMAX)], sidx)
            pltpu.sync_copy(edges_hbm.at[t, 1, pl.ds(NJ * s, NJMAX)], didx)

            for k in range(ROWS_MAIN // ZROWS):
                pltpu.make_async_copy(
                    zbuf, acc.at[pl.ds(base + k * ZROWS, ZROWS)], zsem).wait()

            @pl.when(s == NS - 1)
            def _():
                pltpu.make_async_copy(
                    zbuf.at[pl.ds(0, ROWS_TAIL)],
                    acc.at[pl.ds(ROWS_MAIN * NS, ROWS_TAIL)], zsem).wait()

            plsc.subcore_barrier()

            for r in range(NBUF):
                gather(r, r)

            @pl.loop(0, pl.cdiv(NJ, NBUF))
            def _(kk):
                for r in range(NBUF):
                    j = kk * NBUF + r

                    @pl.when(j < nj)
                    def _():
                        gather_wait(j, r)
                        scatter(j, r)

                    @pl.when(j + NBUF < nj)
                    def _():
                        scatter_wait(j, r)
                        gather(j + NBUF, r)

            for r in range(NBUF):
                scatter_wait(0, r)

            plsc.subcore_barrier()

            pltpu.sync_copy(acc.at[pl.ds(base, ROWS_MAIN)],
                            msgs_hbm.at[t, pl.ds(base, ROWS_MAIN)])

            @pl.when(s == NS - 1)
            def _():
                pltpu.sync_copy(
                    acc.at[pl.ds(ROWS_MAIN * NS, ROWS_TAIL)],
                    msgs_hbm.at[t, pl.ds(ROWS_MAIN * NS, ROWS_TAIL)])

            plsc.subcore_barrier()

    return sc_kernel(node_values, edges_r, zeros)


BV = 2000


def _mm_body(msgs_ref, w_ref, b_ref, out_ref):
    t = pl.program_id(1)

    @pl.when(t == 0)
    def _():
        out_ref[...] = jnp.broadcast_to(b_ref[...], out_ref.shape)

    out_ref[...] += jnp.dot(msgs_ref[0], w_ref[0],
                            preferred_element_type=jnp.float32)


def _tc_matmul(msgs, w3, b2):
    return pl.pallas_call(
        _mm_body,
        grid=(V // BV, T),
        in_specs=[
            pl.BlockSpec((1, BV, D), lambda i, t: (t, i, 0)),
            pl.BlockSpec((1, D, D), lambda i, t: (t, 0, 0)),
            pl.BlockSpec((1, D), lambda i, t: (0, 0)),
        ],
        out_specs=pl.BlockSpec((BV, D), lambda i, t: (i, 0)),
        out_shape=jax.ShapeDtypeStruct((V, D), jnp.float32),
        compiler_params=pltpu.CompilerParams(
            dimension_semantics=("parallel", "arbitrary")),
    )(msgs, w3, b2)


def kernel(node_values, edges, W, b):
    edges_r = edges.astype(jnp.int32).reshape(T, 2, NCHUNKS, 1, CHUNK)
    zeros = jnp.zeros((ZROWS, D), jnp.float32)
    msgs = _sc_message_passing(node_values, edges_r, zeros)
    return _tc_matmul(msgs, W.reshape(T, D, D), b.reshape(1, D))


# speedup vs baseline: 10.8189x; 1.1222x over previous
import functools

import jax
import jax.numpy as jnp
from jax import lax
from jax.experimental import pallas as pl
from jax.experimental.pallas import tpu as pltpu
from jax.experimental.pallas import tpu_sc as plsc

V = 10000
D = 128
T = 4
E = 80000

NC = 2
NS = 16
CHUNK = 128
NCHUNKS = E // CHUNK
TYPES_PER_SC = T // NC
NJ = NCHUNKS // NS
NJMAX = NJ + 1
NBUF = 2
ZROWS = 52

ROWS_MAIN = 624
ROWS_TAIL = V - ROWS_MAIN * NS


def _sc_message_passing(node_values, edges_r, zeros):
    mesh = plsc.VectorSubcoreMesh(core_axis_name="c", subcore_axis_name="s")

    @functools.partial(
        pl.kernel,
        out_type=jax.ShapeDtypeStruct((V, T * D), jnp.float32),
        mesh=mesh,
        scratch_types=[
            pltpu.VMEM_SHARED((V, D), jnp.float32),
            pltpu.VMEM((NJMAX * CHUNK,), jnp.int32),
            pltpu.VMEM((NJMAX * CHUNK,), jnp.int32),
            pltpu.VMEM((NBUF, CHUNK, D), jnp.float32),
            pltpu.VMEM((ZROWS, D), jnp.float32),
            pltpu.SemaphoreType.DMA((NBUF,)),
            pltpu.SemaphoreType.DMA((NBUF,)),
            pltpu.SemaphoreType.DMA,
        ],
    )
    def sc_kernel(node_hbm, edges_hbm, zeros_hbm, msgs_hbm, acc, sidx, didx,
                  rows, zbuf, gsem, ssem, zsem):
        c = lax.axis_index("c")
        s = lax.axis_index("s")
        base = s * ROWS_MAIN
        nj = jnp.where(s == NS - 1, NJMAX, NJ)

        pltpu.sync_copy(zeros_hbm, zbuf)

        def gather(j, r):
            pltpu.async_copy(node_hbm.at[sidx.at[pl.ds(j * CHUNK, CHUNK)]],
                             rows.at[r], gsem.at[r])

        def gather_wait(j, r):
            pltpu.make_async_copy(
                node_hbm.at[sidx.at[pl.ds(j * CHUNK, CHUNK)]],
                rows.at[r], gsem.at[r]).wait()

        def scatter(j, r):
            pltpu.async_copy(rows.at[r],
                             acc.at[didx.at[pl.ds(j * CHUNK, CHUNK)]],
                             ssem.at[r], add=True)

        def scatter_wait(j, r):
            pltpu.make_async_copy(
                rows.at[r], acc.at[didx.at[pl.ds(j * CHUNK, CHUNK)]],
                ssem.at[r]).wait()

        for tt in range(TYPES_PER_SC):
            t = c * TYPES_PER_SC + tt

            for k in range(ROWS_MAIN // ZROWS):
                pltpu.async_copy(zbuf, acc.at[pl.ds(base + k * ZROWS, ZROWS)],
                                 zsem)

            @pl.when(s == NS - 1)
            def _():
                pltpu.async_copy(zbuf.at[pl.ds(0, ROWS_TAIL)],
                                 acc.at[pl.ds(ROWS_MAIN * NS, ROWS_TAIL)],
                                 zsem)

            pltpu.sync_copy(
                edges_hbm.at[t, 0, pl.ds(NJ * s * CHUNK, NJMAX * CHUNK)],
                sidx)
            pltpu.sync_copy(
                edges_hbm.at[t, 1, pl.ds(NJ * s * CHUNK, NJMAX * CHUNK)],
                didx)

            for k in range(ROWS_MAIN // ZROWS):
                pltpu.make_async_copy(
                    zbuf, acc.at[pl.ds(base + k * ZROWS, ZROWS)], zsem).wait()

            @pl.when(s == NS - 1)
            def _():
                pltpu.make_async_copy(
                    zbuf.at[pl.ds(0, ROWS_TAIL)],
                    acc.at[pl.ds(ROWS_MAIN * NS, ROWS_TAIL)], zsem).wait()

            plsc.subcore_barrier()

            for r in range(NBUF):
                gather(r, r)

            @pl.loop(0, pl.cdiv(NJMAX, NBUF))
            def _(kk):
                for r in range(NBUF):
                    j = kk * NBUF + r

                    @pl.when(j < nj)
                    def _():
                        gather_wait(j, r)
                        scatter(j, r)

                    @pl.when(j + NBUF < nj)
                    def _():
                        scatter_wait(j, r)
                        gather(j + NBUF, r)

            for r in range(NBUF):
                scatter_wait(0, r)

            plsc.subcore_barrier()

            pltpu.sync_copy(acc.at[pl.ds(base, ROWS_MAIN)],
                            msgs_hbm.at[pl.ds(base, ROWS_MAIN),
                                        pl.ds(t * D, D)])

            @pl.when(s == NS - 1)
            def _():
                pltpu.sync_copy(
                    acc.at[pl.ds(ROWS_MAIN * NS, ROWS_TAIL)],
                    msgs_hbm.at[pl.ds(ROWS_MAIN * NS, ROWS_TAIL),
                                pl.ds(t * D, D)])

            plsc.subcore_barrier()

    return sc_kernel(node_values, edges_r, zeros)


BV = 2000


def _mm_body(msgs_ref, w_ref, b_ref, out_ref):
    out_ref[...] = b_ref[...] + jnp.dot(msgs_ref[...], w_ref[...],
                                        preferred_element_type=jnp.float32)


def _tc_matmul(msgs, W, b2):
    return pl.pallas_call(
        _mm_body,
        grid=(V // BV,),
        in_specs=[
            pl.BlockSpec((BV, T * D), lambda i: (i, 0)),
            pl.BlockSpec((T * D, D), lambda i: (0, 0)),
            pl.BlockSpec((1, D), lambda i: (0, 0)),
        ],
        out_specs=pl.BlockSpec((BV, D), lambda i: (i, 0)),
        out_shape=jax.ShapeDtypeStruct((V, D), jnp.float32),
        compiler_params=pltpu.CompilerParams(
            dimension_semantics=("parallel",)),
    )(msgs, W, b2)


def kernel(node_values, edges, W, b):
    edges_r = edges.astype(jnp.int32)
    zeros = jnp.zeros((ZROWS, D), jnp.float32)
    msgs = _sc_message_passing(node_values, edges_r, zeros)
    return _tc_matmul(msgs, W, b.reshape(1, D))


# speedup vs baseline: 10.8195x; 1.0001x over previous
import functools

import jax
import jax.numpy as jnp
from jax import lax
from jax.experimental import pallas as pl
from jax.experimental.pallas import tpu as pltpu
from jax.experimental.pallas import tpu_sc as plsc

V = 10000
D = 128
T = 4
E = 80000

NC = 2
NS = 16
CHUNK = 128
NCHUNKS = E // CHUNK
TYPES_PER_SC = T // NC
NJ = NCHUNKS // NS
NJMAX = NJ + 1
NBUF = 2
ZROWS = 52

ROWS_MAIN = 624
ROWS_TAIL = V - ROWS_MAIN * NS


def _sc_message_passing(node_values, edges_r, zeros):
    mesh = plsc.VectorSubcoreMesh(core_axis_name="c", subcore_axis_name="s")

    @functools.partial(
        pl.kernel,
        out_type=jax.ShapeDtypeStruct((V, T * D), jnp.float32),
        mesh=mesh,
        scratch_types=[
            pltpu.VMEM_SHARED((V, D), jnp.float32),
            pltpu.VMEM((NJMAX * CHUNK,), jnp.int32),
            pltpu.VMEM((NJMAX * CHUNK,), jnp.int32),
            pltpu.VMEM((NBUF, CHUNK, D), jnp.float32),
            pltpu.VMEM((ZROWS, D), jnp.float32),
            pltpu.SemaphoreType.DMA((NBUF,)),
            pltpu.SemaphoreType.DMA((NBUF,)),
            pltpu.SemaphoreType.DMA,
        ],
    )
    def sc_kernel(node_hbm, edges_hbm, zeros_hbm, msgs_hbm, acc, sidx, didx,
                  rows, zbuf, gsem, ssem, zsem):
        c = lax.axis_index("c")
        s = lax.axis_index("s")
        base = s * ROWS_MAIN
        nj = jnp.where(s == NS - 1, NJMAX, NJ)

        pltpu.sync_copy(zeros_hbm, zbuf)

        def gather(j, r):
            pltpu.async_copy(node_hbm.at[sidx.at[pl.ds(j * CHUNK, CHUNK)]],
                             rows.at[r], gsem.at[r])

        def gather_wait(j, r):
            pltpu.make_async_copy(
                node_hbm.at[sidx.at[pl.ds(j * CHUNK, CHUNK)]],
                rows.at[r], gsem.at[r]).wait()

        def scatter(j, r):
            pltpu.async_copy(rows.at[r],
                             acc.at[didx.at[pl.ds(j * CHUNK, CHUNK)]],
                             ssem.at[r], add=True)

        def scatter_wait(j, r):
            pltpu.make_async_copy(
                rows.at[r], acc.at[didx.at[pl.ds(j * CHUNK, CHUNK)]],
                ssem.at[r]).wait()

        for tt in range(TYPES_PER_SC):
            t = c * TYPES_PER_SC + tt

            for k in range(ROWS_MAIN // ZROWS):
                pltpu.async_copy(zbuf, acc.at[pl.ds(base + k * ZROWS, ZROWS)],
                                 zsem)

            @pl.when(s == NS - 1)
            def _():
                pltpu.async_copy(zbuf.at[pl.ds(0, ROWS_TAIL)],
                                 acc.at[pl.ds(ROWS_MAIN * NS, ROWS_TAIL)],
                                 zsem)

            pltpu.async_copy(
                edges_hbm.at[t, 0, pl.ds(NJ * s * CHUNK, NJMAX * CHUNK)],
                sidx, gsem.at[0])
            pltpu.async_copy(
                edges_hbm.at[t, 1, pl.ds(NJ * s * CHUNK, NJMAX * CHUNK)],
                didx, gsem.at[1])
            pltpu.make_async_copy(
                edges_hbm.at[t, 0, pl.ds(NJ * s * CHUNK, NJMAX * CHUNK)],
                sidx, gsem.at[0]).wait()
            pltpu.make_async_copy(
                edges_hbm.at[t, 1, pl.ds(NJ * s * CHUNK, NJMAX * CHUNK)],
                didx, gsem.at[1]).wait()

            for k in range(ROWS_MAIN // ZROWS):
                pltpu.make_async_copy(
                    zbuf, acc.at[pl.ds(base + k * ZROWS, ZROWS)], zsem).wait()

            @pl.when(s == NS - 1)
            def _():
                pltpu.make_async_copy(
                    zbuf.at[pl.ds(0, ROWS_TAIL)],
                    acc.at[pl.ds(ROWS_MAIN * NS, ROWS_TAIL)], zsem).wait()

            plsc.subcore_barrier()

            for r in range(NBUF):
                gather(r, r)

            @pl.loop(0, pl.cdiv(NJMAX, NBUF))
            def _(kk):
                for r in range(NBUF):
                    j = kk * NBUF + r

                    @pl.when(j < nj)
                    def _():
                        gather_wait(j, r)
                        scatter(j, r)

                    @pl.when(j + NBUF < nj)
                    def _():
                        scatter_wait(j, r)
                        gather(j + NBUF, r)

            for r in range(NBUF):
                scatter_wait(0, r)

            plsc.subcore_barrier()

            pltpu.sync_copy(acc.at[pl.ds(base, ROWS_MAIN)],
                            msgs_hbm.at[pl.ds(base, ROWS_MAIN),
                                        pl.ds(t * D, D)])

            @pl.when(s == NS - 1)
            def _():
                pltpu.sync_copy(
                    acc.at[pl.ds(ROWS_MAIN * NS, ROWS_TAIL)],
                    msgs_hbm.at[pl.ds(ROWS_MAIN * NS, ROWS_TAIL),
                                pl.ds(t * D, D)])


    return sc_kernel(node_values, edges_r, zeros)


BV = 2000


def _mm_body(msgs_ref, w_ref, b_ref, out_ref):
    out_ref[...] = b_ref[...] + jnp.dot(msgs_ref[...], w_ref[...],
                                        preferred_element_type=jnp.float32)


def _tc_matmul(msgs, W, b2):
    return pl.pallas_call(
        _mm_body,
        grid=(V // BV,),
        in_specs=[
            pl.BlockSpec((BV, T * D), lambda i: (i, 0)),
            pl.BlockSpec((T * D, D), lambda i: (0, 0)),
            pl.BlockSpec((1, D), lambda i: (0, 0)),
        ],
        out_specs=pl.BlockSpec((BV, D), lambda i: (i, 0)),
        out_shape=jax.ShapeDtypeStruct((V, D), jnp.float32),
        compiler_params=pltpu.CompilerParams(
            dimension_semantics=("parallel",)),
    )(msgs, W, b2)


def kernel(node_values, edges, W, b):
    edges_r = edges.astype(jnp.int32)
    zeros = jnp.zeros((ZROWS, D), jnp.float32)
    msgs = _sc_message_passing(node_values, edges_r, zeros)
    return _tc_matmul(msgs, W, b.reshape(1, D))


# speedup vs baseline: 11.8456x; 1.0948x over previous
import functools

import jax
import jax.numpy as jnp
from jax import lax
from jax.experimental import pallas as pl
from jax.experimental.pallas import tpu as pltpu
from jax.experimental.pallas import tpu_sc as plsc

V = 10000
D = 128
T = 4
E = 80000

NC = 2
NS = 16
CHUNK = 40
TYPES_PER_SC = T // NC
EPT = E // NS
NJ = EPT // CHUNK
NBUF = 5
STAGE_OFF = 4992
STAGE_LEN = 5120
ZROWS = 48

ROWS_MAIN = 624
ROWS_TAIL = V - ROWS_MAIN * NS


def _sc_message_passing(node_values, edges_r, zeros):
    mesh = plsc.VectorSubcoreMesh(core_axis_name="c", subcore_axis_name="s")

    @functools.partial(
        pl.kernel,
        out_type=jax.ShapeDtypeStruct((V, T * D), jnp.float32),
        mesh=mesh,
        scratch_types=[
            pltpu.VMEM_SHARED((V, D), jnp.float32),
            pltpu.VMEM((STAGE_LEN,), jnp.int32),
            pltpu.VMEM((STAGE_LEN,), jnp.int32),
            pltpu.VMEM((NBUF, CHUNK, D), jnp.float32),
            pltpu.VMEM((ZROWS, D), jnp.float32),
            pltpu.SemaphoreType.DMA((NBUF,)),
            pltpu.SemaphoreType.DMA((NBUF,)),
            pltpu.SemaphoreType.DMA,
        ],
    )
    def sc_kernel(node_hbm, edges_hbm, zeros_hbm, msgs_hbm, acc, sidx, didx,
                  rows, zbuf, gsem, ssem, zsem):
        c = lax.axis_index("c")
        s = lax.axis_index("s")
        base = s * ROWS_MAIN
        ioff = 8 * s

        pltpu.sync_copy(zeros_hbm, zbuf)

        def gather(j, r):
            pltpu.async_copy(
                node_hbm.at[sidx.at[pl.ds(ioff + j * CHUNK, CHUNK)]],
                rows.at[r], gsem.at[r])

        def gather_wait(j, r):
            pltpu.make_async_copy(
                node_hbm.at[sidx.at[pl.ds(ioff + j * CHUNK, CHUNK)]],
                rows.at[r], gsem.at[r]).wait()

        def scatter(j, r):
            pltpu.async_copy(rows.at[r],
                             acc.at[didx.at[pl.ds(ioff + j * CHUNK, CHUNK)]],
                             ssem.at[r], add=True)

        def scatter_wait(j, r):
            pltpu.make_async_copy(
                rows.at[r], acc.at[didx.at[pl.ds(ioff + j * CHUNK, CHUNK)]],
                ssem.at[r]).wait()

        for tt in range(TYPES_PER_SC):
            t = c * TYPES_PER_SC + tt

            for k in range(ROWS_MAIN // ZROWS):
                pltpu.async_copy(zbuf, acc.at[pl.ds(base + k * ZROWS, ZROWS)],
                                 zsem)

            @pl.when(s == NS - 1)
            def _():
                pltpu.async_copy(zbuf.at[pl.ds(0, ROWS_TAIL)],
                                 acc.at[pl.ds(ROWS_MAIN * NS, ROWS_TAIL)],
                                 zsem)

            pltpu.async_copy(
                edges_hbm.at[t, 0, pl.ds(STAGE_OFF * s, STAGE_LEN)],
                sidx, gsem.at[0])
            pltpu.async_copy(
                edges_hbm.at[t, 1, pl.ds(STAGE_OFF * s, STAGE_LEN)],
                didx, gsem.at[1])
            pltpu.make_async_copy(
                edges_hbm.at[t, 0, pl.ds(STAGE_OFF * s, STAGE_LEN)],
                sidx, gsem.at[0]).wait()
            pltpu.make_async_copy(
                edges_hbm.at[t, 1, pl.ds(STAGE_OFF * s, STAGE_LEN)],
                didx, gsem.at[1]).wait()

            for k in range(ROWS_MAIN // ZROWS):
                pltpu.make_async_copy(
                    zbuf, acc.at[pl.ds(base + k * ZROWS, ZROWS)], zsem).wait()

            @pl.when(s == NS - 1)
            def _():
                pltpu.make_async_copy(
                    zbuf.at[pl.ds(0, ROWS_TAIL)],
                    acc.at[pl.ds(ROWS_MAIN * NS, ROWS_TAIL)], zsem).wait()

            plsc.subcore_barrier()

            for r in range(NBUF):
                gather(r, r)

            @pl.loop(0, NJ // NBUF)
            def _(kk):
                for r in range(NBUF):
                    j = kk * NBUF + r
                    gather_wait(j, r)
                    scatter(j, r)

                    @pl.when(j + NBUF < NJ)
                    def _():
                        scatter_wait(j, r)
                        gather(j + NBUF, r)

            for r in range(NBUF):
                scatter_wait(0, r)

            plsc.subcore_barrier()

            pltpu.sync_copy(acc.at[pl.ds(base, ROWS_MAIN)],
                            msgs_hbm.at[pl.ds(base, ROWS_MAIN),
                                        pl.ds(t * D, D)])

            @pl.when(s == NS - 1)
            def _():
                pltpu.sync_copy(
                    acc.at[pl.ds(ROWS_MAIN * NS, ROWS_TAIL)],
                    msgs_hbm.at[pl.ds(ROWS_MAIN * NS, ROWS_TAIL),
                                pl.ds(t * D, D)])


    return sc_kernel(node_values, edges_r, zeros)


BV = 2000


def _mm_body(msgs_ref, w_ref, b_ref, out_ref):
    out_ref[...] = b_ref[...] + jnp.dot(msgs_ref[...], w_ref[...],
                                        preferred_element_type=jnp.float32)


def _tc_matmul(msgs, W, b2):
    return pl.pallas_call(
        _mm_body,
        grid=(V // BV,),
        in_specs=[
            pl.BlockSpec((BV, T * D), lambda i: (i, 0)),
            pl.BlockSpec((T * D, D), lambda i: (0, 0)),
            pl.BlockSpec((1, D), lambda i: (0, 0)),
        ],
        out_specs=pl.BlockSpec((BV, D), lambda i: (i, 0)),
        out_shape=jax.ShapeDtypeStruct((V, D), jnp.float32),
        compiler_params=pltpu.CompilerParams(
            dimension_semantics=("parallel",)),
    )(msgs, W, b2)


def kernel(node_values, edges, W, b):
    edges_r = edges.astype(jnp.int32)
    zeros = jnp.zeros((ZROWS, D), jnp.float32)
    msgs = _sc_message_passing(node_values, edges_r, zeros)
    return _tc_matmul(msgs, W, b.reshape(1, D))


# speedup vs baseline: 11.9056x; 1.0051x over previous
import functools

import jax
import jax.numpy as jnp
from jax import lax
from jax.experimental import pallas as pl
from jax.experimental.pallas import tpu as pltpu
from jax.experimental.pallas import tpu_sc as plsc

V = 10000
D = 128
T = 4
E = 80000

NC = 2
NS = 16
CHUNK = 40
TYPES_PER_SC = T // NC
EPT = E // NS
NJ = EPT // CHUNK
NBUF = 6
STAGE_OFF = 4992
STAGE_LEN = 5120
ZROWS = 48

ROWS_MAIN = 624
ROWS_TAIL = V - ROWS_MAIN * NS


def _sc_message_passing(node_values, edges_r, zeros):
    mesh = plsc.VectorSubcoreMesh(core_axis_name="c", subcore_axis_name="s")

    @functools.partial(
        pl.kernel,
        out_type=jax.ShapeDtypeStruct((V, T * D), jnp.float32),
        mesh=mesh,
        scratch_types=[
            pltpu.VMEM_SHARED((V, D), jnp.float32),
            pltpu.VMEM((STAGE_LEN,), jnp.int32),
            pltpu.VMEM((STAGE_LEN,), jnp.int32),
            pltpu.VMEM((NBUF, CHUNK, D), jnp.float32),
            pltpu.VMEM((ZROWS, D), jnp.float32),
            pltpu.SemaphoreType.DMA((NBUF,)),
            pltpu.SemaphoreType.DMA((NBUF,)),
            pltpu.SemaphoreType.DMA,
        ],
    )
    def sc_kernel(node_hbm, edges_hbm, zeros_hbm, msgs_hbm, acc, sidx, didx,
                  rows, zbuf, gsem, ssem, zsem):
        c = lax.axis_index("c")
        s = lax.axis_index("s")
        base = s * ROWS_MAIN
        ioff = 8 * s

        pltpu.sync_copy(zeros_hbm, zbuf)

        def gather(j, r):
            pltpu.async_copy(
                node_hbm.at[sidx.at[pl.ds(ioff + j * CHUNK, CHUNK)]],
                rows.at[r], gsem.at[r])

        def gather_wait(j, r):
            pltpu.make_async_copy(
                node_hbm.at[sidx.at[pl.ds(ioff + j * CHUNK, CHUNK)]],
                rows.at[r], gsem.at[r]).wait()

        def scatter(j, r):
            pltpu.async_copy(rows.at[r],
                             acc.at[didx.at[pl.ds(ioff + j * CHUNK, CHUNK)]],
                             ssem.at[r], add=True)

        def scatter_wait(j, r):
            pltpu.make_async_copy(
                rows.at[r], acc.at[didx.at[pl.ds(ioff + j * CHUNK, CHUNK)]],
                ssem.at[r]).wait()

        for tt in range(TYPES_PER_SC):
            t = c * TYPES_PER_SC + tt

            for k in range(ROWS_MAIN // ZROWS):
                pltpu.async_copy(zbuf, acc.at[pl.ds(base + k * ZROWS, ZROWS)],
                                 zsem)

            @pl.when(s == NS - 1)
            def _():
                pltpu.async_copy(zbuf.at[pl.ds(0, ROWS_TAIL)],
                                 acc.at[pl.ds(ROWS_MAIN * NS, ROWS_TAIL)],
                                 zsem)

            pltpu.async_copy(
                edges_hbm.at[t, 0, pl.ds(STAGE_OFF * s, STAGE_LEN)],
                sidx, gsem.at[0])
            pltpu.async_copy(
                edges_hbm.at[t, 1, pl.ds(STAGE_OFF * s, STAGE_LEN)],
                didx, gsem.at[1])
            pltpu.make_async_copy(
                edges_hbm.at[t, 0, pl.ds(STAGE_OFF * s, STAGE_LEN)],
                sidx, gsem.at[0]).wait()
            pltpu.make_async_copy(
                edges_hbm.at[t, 1, pl.ds(STAGE_OFF * s, STAGE_LEN)],
                didx, gsem.at[1]).wait()

            for k in range(ROWS_MAIN // ZROWS):
                pltpu.make_async_copy(
                    zbuf, acc.at[pl.ds(base + k * ZROWS, ZROWS)], zsem).wait()

            @pl.when(s == NS - 1)
            def _():
                pltpu.make_async_copy(
                    zbuf.at[pl.ds(0, ROWS_TAIL)],
                    acc.at[pl.ds(ROWS_MAIN * NS, ROWS_TAIL)], zsem).wait()

            plsc.subcore_barrier()

            for r in range(NBUF):
                gather(r, r)

            @pl.loop(0, pl.cdiv(NJ, NBUF))
            def _(kk):
                for r in range(NBUF):
                    j = kk * NBUF + r

                    @pl.when(j < NJ)
                    def _():
                        gather_wait(j, r)
                        scatter(j, r)

                    @pl.when(j + NBUF < NJ)
                    def _():
                        scatter_wait(j, r)
                        gather(j + NBUF, r)

            for r in range(NBUF):
                scatter_wait(0, r)

            plsc.subcore_barrier()

            pltpu.sync_copy(acc.at[pl.ds(base, ROWS_MAIN)],
                            msgs_hbm.at[pl.ds(base, ROWS_MAIN),
                                        pl.ds(t * D, D)])

            @pl.when(s == NS - 1)
            def _():
                pltpu.sync_copy(
                    acc.at[pl.ds(ROWS_MAIN * NS, ROWS_TAIL)],
                    msgs_hbm.at[pl.ds(ROWS_MAIN * NS, ROWS_TAIL),
                                pl.ds(t * D, D)])


    return sc_kernel(node_values, edges_r, zeros)


BV = 2000


def _mm_body(msgs_ref, w_ref, b_ref, out_ref):
    out_ref[...] = b_ref[...] + jnp.dot(msgs_ref[...], w_ref[...],
                                        preferred_element_type=jnp.float32)


def _tc_matmul(msgs, W, b2):
    return pl.pallas_call(
        _mm_body,
        grid=(V // BV,),
        in_specs=[
            pl.BlockSpec((BV, T * D), lambda i: (i, 0)),
            pl.BlockSpec((T * D, D), lambda i: (0, 0)),
            pl.BlockSpec((1, D), lambda i: (0, 0)),
        ],
        out_specs=pl.BlockSpec((BV, D), lambda i: (i, 0)),
        out_shape=jax.ShapeDtypeStruct((V, D), jnp.float32),
        compiler_params=pltpu.CompilerParams(
            dimension_semantics=("parallel",)),
    )(msgs, W, b2)


def kernel(node_values, edges, W, b):
    edges_r = edges.astype(jnp.int32)
    zeros = jnp.zeros((ZROWS, D), jnp.float32)
    msgs = _sc_message_passing(node_values, edges_r, zeros)
    return _tc_matmul(msgs, W, b.reshape(1, D))
